# Initial kernel scaffold; baseline (speedup 1.0000x reference)
import functools

import jax
import jax.numpy as jnp
from jax import lax
from jax.experimental import pallas as pl
from jax.experimental.pallas import tpu as pltpu
from jax.experimental.pallas import tpu_sc as plsc

_INFO = plsc.get_sparse_core_info()
_NC = _INFO.num_cores
_NS = _INFO.num_subcores
_NW = _NC * _NS
_K = 80


@functools.partial(jax.jit, static_argnums=(0, 1, 2))
def _sc_aggregate(n_nodes, n_edges, d, x, src, dst, zeros):
    edges_per_worker = n_edges // _NW
    steps = edges_per_worker // _K
    rows_per_tile = n_nodes // _NS

    mesh = plsc.VectorSubcoreMesh(core_axis_name="c", subcore_axis_name="s")

    @functools.partial(
        pl.kernel,
        out_type=jax.ShapeDtypeStruct((_NC * n_nodes, d), jnp.float32),
        mesh=mesh,
        scratch_types=[
            pltpu.VMEM((_K,), jnp.int32),
            pltpu.VMEM((_K,), jnp.int32),
            pltpu.VMEM((_K, d), jnp.float32),
            pltpu.VMEM_SHARED((n_nodes, d), jnp.float32),
            pltpu.SemaphoreType.DMA,
        ],
    )
    def agg_kernel(x_hbm, src_hbm, dst_hbm, zeros_hbm, part_hbm,
                   src_v, dst_v, rows_v, acc_sh, sem):
        c = lax.axis_index("c")
        s = lax.axis_index("s")
        wid = s * _NC + c

        pltpu.sync_copy(zeros_hbm.at[pl.ds(s * rows_per_tile, rows_per_tile)],
                        acc_sh.at[pl.ds(s * rows_per_tile, rows_per_tile)])
        plsc.subcore_barrier()

        base0 = wid * edges_per_worker

        def step(i, carry):
            base = base0 + i * _K
            pltpu.sync_copy(src_hbm.at[pl.ds(base, _K)], src_v)
            pltpu.sync_copy(dst_hbm.at[pl.ds(base, _K)], dst_v)
            pltpu.async_copy(x_hbm.at[src_v], rows_v, sem).wait()
            pltpu.sync_copy(rows_v, acc_sh.at[dst_v], add=True)
            return carry

        lax.fori_loop(0, steps, step, 0)
        plsc.subcore_barrier()

        off = c * n_nodes + s * rows_per_tile
        pltpu.sync_copy(acc_sh.at[pl.ds(s * rows_per_tile, rows_per_tile)],
                        part_hbm.at[pl.ds(off, rows_per_tile)])

    return agg_kernel(x, src, dst, zeros)


def _tc_finish_body(p_ref, x_ref, w_ref, slwt_ref, o_ref):
    agg = p_ref[0] + p_ref[1]
    o_ref[...] = (
        jnp.dot(agg, w_ref[...], preferred_element_type=jnp.float32)
        + jnp.dot(x_ref[...], slwt_ref[...], preferred_element_type=jnp.float32)
    )


def kernel(x, edge_index, edge_type, weight, self_loop_w):
    n_nodes, d = x.shape
    n_edges = edge_index.shape[1]
    src = edge_index[0]
    dst = edge_index[1]
    zeros = jnp.zeros((n_nodes, d), jnp.float32)

    part = _sc_aggregate(n_nodes, n_edges, d, x, src, dst, zeros)
    part3 = part.reshape(_NC, n_nodes, d)

    out = pl.pallas_call(
        _tc_finish_body,
        out_shape=jax.ShapeDtypeStruct((n_nodes, d), jnp.float32),
    )(part3, x, weight[0], self_loop_w.T)
    return out


# speedup vs baseline: 5.2403x; 5.2403x over previous
import functools

import jax
import jax.numpy as jnp
from jax import lax
from jax.experimental import pallas as pl
from jax.experimental.pallas import tpu as pltpu
from jax.experimental.pallas import tpu_sc as plsc

_INFO = plsc.get_sparse_core_info()
_NC = _INFO.num_cores
_NS = _INFO.num_subcores
_NW = _NC * _NS
_K = 80


@functools.partial(jax.jit, static_argnums=(0, 1, 2))
def _sc_aggregate(n_pad, n_edges, d, x, src, dst, zeros):
    edges_per_worker = n_edges // _NW
    steps = edges_per_worker // _K
    rows_per_tile = n_pad // _NS

    mesh = plsc.VectorSubcoreMesh(core_axis_name="c", subcore_axis_name="s")

    @functools.partial(
        pl.kernel,
        out_type=jax.ShapeDtypeStruct((_NC * n_pad, d), jnp.float32),
        mesh=mesh,
        scratch_types=[
            pltpu.VMEM((_K,), jnp.int32),
            pltpu.VMEM((_K,), jnp.int32),
            pltpu.VMEM((_K, d), jnp.float32),
            pltpu.VMEM_SHARED((n_pad, d), jnp.float32),
            pltpu.SemaphoreType.DMA,
        ],
    )
    def agg_kernel(x_hbm, src_hbm, dst_hbm, zeros_hbm, part_hbm,
                   src_v, dst_v, rows_v, acc_sh, sem):
        c = lax.axis_index("c")
        s = lax.axis_index("s")
        wid = s * _NC + c

        pltpu.sync_copy(zeros_hbm.at[pl.ds(s * rows_per_tile, rows_per_tile)],
                        acc_sh.at[pl.ds(s * rows_per_tile, rows_per_tile)])
        plsc.subcore_barrier()

        base0 = wid * edges_per_worker

        def step(i, carry):
            base = base0 + i * _K
            pltpu.sync_copy(src_hbm.at[pl.ds(base, _K)], src_v)
            pltpu.sync_copy(dst_hbm.at[pl.ds(base, _K)], dst_v)
            pltpu.async_copy(x_hbm.at[src_v], rows_v, sem).wait()
            pltpu.sync_copy(rows_v, acc_sh.at[dst_v], add=True)
            return carry

        lax.fori_loop(0, steps, step, 0)
        plsc.subcore_barrier()

        off = c * n_pad + s * rows_per_tile
        pltpu.sync_copy(acc_sh.at[pl.ds(s * rows_per_tile, rows_per_tile)],
                        part_hbm.at[pl.ds(off, rows_per_tile)])

    return agg_kernel(x, src, dst, zeros)


def _tc_finish_body(p0_ref, p1_ref, x_ref, w_ref, slwt_ref, o_ref):
    agg = p0_ref[0] + p1_ref[0]
    o_ref[...] = (
        jnp.dot(agg, w_ref[...], preferred_element_type=jnp.float32)
        + jnp.dot(x_ref[...], slwt_ref[...], preferred_element_type=jnp.float32)
    )


def kernel(x, edge_index, edge_type, weight, self_loop_w):
    n_nodes, d = x.shape
    n_edges = edge_index.shape[1]
    n_pad = ((n_nodes + 8 * _NS - 1) // (8 * _NS)) * (8 * _NS)
    src = edge_index[0]
    dst = edge_index[1]
    zeros = jnp.zeros((n_pad, d), jnp.float32)

    part = _sc_aggregate(n_pad, n_edges, d, x, src, dst, zeros)
    part3 = part.reshape(_NC, n_pad, d)

    blk = 2000
    grid = n_nodes // blk
    out = pl.pallas_call(
        _tc_finish_body,
        grid=(grid,),
        in_specs=[
            pl.BlockSpec((1, blk, d), lambda i: (0, i, 0)),
            pl.BlockSpec((1, blk, d), lambda i: (1, i, 0)),
            pl.BlockSpec((blk, d), lambda i: (i, 0)),
            pl.BlockSpec((d, d), lambda i: (0, 0)),
            pl.BlockSpec((d, d), lambda i: (0, 0)),
        ],
        out_specs=pl.BlockSpec((blk, d), lambda i: (i, 0)),
        out_shape=jax.ShapeDtypeStruct((n_nodes, d), jnp.float32),
    )(part3, part3, x, weight[0], self_loop_w.T)
    return out


# speedup vs baseline: 11.9063x; 2.2721x over previous
import functools

import jax
import jax.numpy as jnp
from jax import lax
from jax.experimental import pallas as pl
from jax.experimental.pallas import tpu as pltpu
from jax.experimental.pallas import tpu_sc as plsc

_INFO = plsc.get_sparse_core_info()
_NC = _INFO.num_cores
_NS = _INFO.num_subcores
_NW = _NC * _NS
_K = 125


@functools.partial(jax.jit, static_argnums=(0, 1, 2))
def _sc_aggregate(n_pad, n_edges, d, x, src2, dst2, zeros):
    edges_per_worker = n_edges // _NW
    steps = edges_per_worker // _K
    n_phases = 2
    hs = steps // n_phases
    rows_per_tile = n_pad // _NS

    mesh = plsc.VectorSubcoreMesh(core_axis_name="c", subcore_axis_name="s")

    @functools.partial(
        pl.kernel,
        out_type=jax.ShapeDtypeStruct((_NC * n_pad, d), jnp.float32),
        mesh=mesh,
        scratch_types=[
            pltpu.VMEM((hs, _K), jnp.int32),
            pltpu.VMEM((hs, _K), jnp.int32),
            pltpu.VMEM((_K, d), jnp.float32),
            pltpu.VMEM((_K, d), jnp.float32),
            pltpu.VMEM_SHARED((n_pad, d), jnp.float32),
            pltpu.SemaphoreType.DMA,
            pltpu.SemaphoreType.DMA,
            pltpu.SemaphoreType.DMA,
        ],
    )
    def agg_kernel(x_hbm, src_hbm, dst_hbm, zeros_hbm, part_hbm,
                   src_v, dst_v, rows_a, rows_b, acc_sh,
                   sem_i, sem_a, sem_b):
        c = lax.axis_index("c")
        s = lax.axis_index("s")
        wid = s * _NC + c
        row0 = wid * steps

        def gather(i, buf, sem):
            return pltpu.async_copy(x_hbm.at[src_v.at[i]], buf, sem)

        def scat(i, buf):
            pltpu.sync_copy(buf, acc_sh.at[dst_v.at[i]], add=True)

        for p in range(n_phases):
            r0 = row0 + p * hs
            cp_src = pltpu.async_copy(src_hbm.at[pl.ds(r0, hs)], src_v, sem_i)
            cp_dst = pltpu.async_copy(dst_hbm.at[pl.ds(r0, hs)], dst_v, sem_i)
            if p == 0:
                pltpu.sync_copy(
                    zeros_hbm.at[pl.ds(s * rows_per_tile, rows_per_tile)],
                    acc_sh.at[pl.ds(s * rows_per_tile, rows_per_tile)])
            cp_src.wait()
            cp_dst.wait()
            if p == 0:
                plsc.subcore_barrier()

            gather(0, rows_a, sem_a)

            def body(j, carry):
                i = 2 * j
                gather(i + 1, rows_b, sem_b)
                pltpu.make_async_copy(x_hbm.at[src_v.at[i]], rows_a, sem_a).wait()
                scat(i, rows_a)

                @pl.when(j < hs // 2 - 1)
                def _():
                    gather(i + 2, rows_a, sem_a)

                pltpu.make_async_copy(x_hbm.at[src_v.at[i + 1]], rows_b, sem_b).wait()
                scat(i + 1, rows_b)
                return carry

            lax.fori_loop(0, hs // 2, body, 0)
        plsc.subcore_barrier()

        off = c * n_pad + s * rows_per_tile
        pltpu.sync_copy(acc_sh.at[pl.ds(s * rows_per_tile, rows_per_tile)],
                        part_hbm.at[pl.ds(off, rows_per_tile)])

    return agg_kernel(x, src2, dst2, zeros)


def _tc_finish_body(p0_ref, p1_ref, x_ref, w_ref, slwt_ref, o_ref):
    agg = p0_ref[0] + p1_ref[0]
    o_ref[...] = (
        jnp.dot(agg, w_ref[...], preferred_element_type=jnp.float32)
        + jnp.dot(x_ref[...], slwt_ref[...], preferred_element_type=jnp.float32)
    )


def kernel(x, edge_index, edge_type, weight, self_loop_w):
    n_nodes, d = x.shape
    n_edges = edge_index.shape[1]
    n_pad = ((n_nodes + 8 * _NS - 1) // (8 * _NS)) * (8 * _NS)
    src2 = edge_index[0].reshape(n_edges // _K, _K)
    dst2 = edge_index[1].reshape(n_edges // _K, _K)
    zeros = jnp.zeros((n_pad, d), jnp.float32)

    part = _sc_aggregate(n_pad, n_edges, d, x, src2, dst2, zeros)
    part3 = part.reshape(_NC, n_pad, d)

    blk = 2000
    grid = n_nodes // blk
    out = pl.pallas_call(
        _tc_finish_body,
        grid=(grid,),
        in_specs=[
            pl.BlockSpec((1, blk, d), lambda i: (0, i, 0)),
            pl.BlockSpec((1, blk, d), lambda i: (1, i, 0)),
            pl.BlockSpec((blk, d), lambda i: (i, 0)),
            pl.BlockSpec((d, d), lambda i: (0, 0)),
            pl.BlockSpec((d, d), lambda i: (0, 0)),
        ],
        out_specs=pl.BlockSpec((blk, d), lambda i: (i, 0)),
        out_shape=jax.ShapeDtypeStruct((n_nodes, d), jnp.float32),
    )(part3, part3, x, weight[0], self_loop_w.T)
    return out


# speedup vs baseline: 12.7026x; 1.0669x over previous
import functools

import jax
import jax.numpy as jnp
from jax import lax
from jax.experimental import pallas as pl
from jax.experimental.pallas import tpu as pltpu
from jax.experimental.pallas import tpu_sc as plsc

_INFO = plsc.get_sparse_core_info()
_NC = _INFO.num_cores
_NS = _INFO.num_subcores
_NW = _NC * _NS
_K = 125


@functools.partial(jax.jit, static_argnums=(0, 1, 2))
def _sc_aggregate(n_pad, n_edges, d, x, eidx3, zeros):
    edges_per_worker = n_edges // _NW
    steps = edges_per_worker // _K
    n_phases = 2
    hs = steps // n_phases
    rows_per_tile = n_pad // _NS

    mesh = plsc.VectorSubcoreMesh(core_axis_name="c", subcore_axis_name="s")

    @functools.partial(
        pl.kernel,
        out_type=jax.ShapeDtypeStruct((_NC * n_pad, d), jnp.float32),
        mesh=mesh,
        scratch_types=[
            pltpu.VMEM((hs, _K), jnp.int32),
            pltpu.VMEM((hs, _K), jnp.int32),
            pltpu.VMEM((_K, d), jnp.float32),
            pltpu.VMEM((_K, d), jnp.float32),
            pltpu.VMEM_SHARED((n_pad, d), jnp.float32),
            pltpu.SemaphoreType.DMA,
            pltpu.SemaphoreType.DMA,
            pltpu.SemaphoreType.DMA,
        ],
    )
    def agg_kernel(x_hbm, eidx_hbm, zeros_hbm, part_hbm,
                   src_v, dst_v, rows_a, rows_b, acc_sh,
                   sem_i, sem_a, sem_b):
        c = lax.axis_index("c")
        s = lax.axis_index("s")
        wid = s * _NC + c
        row0 = wid * steps

        def gather(i, buf, sem):
            return pltpu.async_copy(x_hbm.at[src_v.at[i]], buf, sem)

        def scat(i, buf):
            pltpu.sync_copy(buf, acc_sh.at[dst_v.at[i]], add=True)

        for p in range(n_phases):
            r0 = row0 + p * hs
            cp_src = pltpu.async_copy(eidx_hbm.at[0, pl.ds(r0, hs)], src_v,
                                      sem_i)
            cp_dst = pltpu.async_copy(eidx_hbm.at[1, pl.ds(r0, hs)], dst_v,
                                      sem_i)
            if p == 0:
                pltpu.sync_copy(
                    zeros_hbm,
                    acc_sh.at[pl.ds(s * rows_per_tile, rows_per_tile)])
            cp_src.wait()
            cp_dst.wait()
            if p == 0:
                plsc.subcore_barrier()

            gather(0, rows_a, sem_a)

            def body(j, carry):
                i = 2 * j
                gather(i + 1, rows_b, sem_b)
                pltpu.make_async_copy(x_hbm.at[src_v.at[i]], rows_a,
                                      sem_a).wait()
                scat(i, rows_a)

                @pl.when(j < hs // 2 - 1)
                def _():
                    gather(i + 2, rows_a, sem_a)

                pltpu.make_async_copy(x_hbm.at[src_v.at[i + 1]], rows_b,
                                      sem_b).wait()
                scat(i + 1, rows_b)
                return carry

            lax.fori_loop(0, hs // 2, body, 0)
        plsc.subcore_barrier()

        off = c * n_pad + s * rows_per_tile
        pltpu.sync_copy(acc_sh.at[pl.ds(s * rows_per_tile, rows_per_tile)],
                        part_hbm.at[pl.ds(off, rows_per_tile)])

    return agg_kernel(x, eidx3, zeros)


def _tc_selfloop_body(x_ref, slw_ref, o_ref):
    o_ref[...] = lax.dot_general(
        x_ref[...], slw_ref[...], (((1,), (1,)), ((), ())),
        preferred_element_type=jnp.float32)


def _tc_combine_body(p0_ref, p1_ref, sl_ref, w_ref, o_ref):
    agg = p0_ref[0] + p1_ref[0]
    o_ref[...] = (
        jnp.dot(agg, w_ref[...], preferred_element_type=jnp.float32)
        + sl_ref[...]
    )


def kernel(x, edge_index, edge_type, weight, self_loop_w):
    n_nodes, d = x.shape
    n_edges = edge_index.shape[1]
    n_pad = ((n_nodes + 8 * _NS - 1) // (8 * _NS)) * (8 * _NS)
    eidx3 = edge_index.reshape(2, n_edges // _K, _K)
    zeros = jnp.zeros((n_pad // _NS, d), jnp.float32)

    blk = 2000
    grid = n_nodes // blk

    selfloop = pl.pallas_call(
        _tc_selfloop_body,
        grid=(grid,),
        in_specs=[
            pl.BlockSpec((blk, d), lambda i: (i, 0)),
            pl.BlockSpec((d, d), lambda i: (0, 0)),
        ],
        out_specs=pl.BlockSpec((blk, d), lambda i: (i, 0)),
        out_shape=jax.ShapeDtypeStruct((n_nodes, d), jnp.float32),
    )(x, self_loop_w)

    part = _sc_aggregate(n_pad, n_edges, d, x, eidx3, zeros)
    part3 = part.reshape(_NC, n_pad, d)

    out = pl.pallas_call(
        _tc_combine_body,
        grid=(grid,),
        in_specs=[
            pl.BlockSpec((1, blk, d), lambda i: (0, i, 0)),
            pl.BlockSpec((1, blk, d), lambda i: (1, i, 0)),
            pl.BlockSpec((blk, d), lambda i: (i, 0)),
            pl.BlockSpec((d, d), lambda i: (0, 0)),
        ],
        out_specs=pl.BlockSpec((blk, d), lambda i: (i, 0)),
        out_shape=jax.ShapeDtypeStruct((n_nodes, d), jnp.float32),
    )(part3, part3, selfloop, weight[0])
    return out
